# Initial kernel scaffold; baseline (speedup 1.0000x reference)
import jax
import jax.numpy as jnp
from jax.experimental import pallas as pl


def kernel(x, edge_index, batch, params):
    raise NotImplementedError("write your pallas kernel here")


# speedup vs baseline: 16.6883x; 16.6883x over previous
import functools

import jax
import jax.numpy as jnp
import numpy as np
from jax import lax
from jax.experimental import pallas as pl
from jax.experimental.pallas import tpu as pltpu
from jax.experimental.pallas import tpu_sc as plsc

N = 10000
E = 320000
B = 64
F = 128
H = 64

NC = 2
NS = 16
NW = NC * NS
CH = 128
NCHUNK = E // CH
NFULL = NCHUNK // NW
NEXTRA = NCHUNK - NFULL * NW
WCH = 200
NWCH = N // WCH

_HI = jax.lax.Precision.HIGHEST
_F32 = jnp.float32


def _mm(a, b):
    return lax.dot_general(a, b, (((1,), (0,)), ((), ())),
                           precision=_HI, preferred_element_type=_F32)


def _mm11(a, b):
    return lax.dot_general(a, b, (((1,), (1,)), ((), ())),
                           precision=_HI, preferred_element_type=_F32)


def _mm00(a, b):
    return lax.dot_general(a, b, (((0,), (0,)), ((), ())),
                           precision=_HI, preferred_element_type=_F32)


def _ln(x, w, b):
    m = jnp.mean(x, axis=1, keepdims=True)
    xc = x - m
    v = jnp.mean(xc * xc, axis=1, keepdims=True)
    return xc / jnp.sqrt(v + 1e-5) * w + b


def _sc_mesh():
    return plsc.VectorSubcoreMesh(core_axis_name="c", subcore_axis_name="s",
                                  num_cores=NC, num_subcores=NS)


DW = 16


@functools.cache
def _make_sc_degree():
    return functools.partial(
        pl.kernel,
        out_type=jax.ShapeDtypeStruct((NC, N, DW), _F32),
        mesh=_sc_mesh(),
        scratch_types=[
            pltpu.VMEM((2, CH), jnp.int32),
            pltpu.VMEM((CH, DW), _F32),
            pltpu.VMEM((WCH, DW), _F32),
            pltpu.VMEM_SHARED((N, DW), _F32),
        ],
    )(_sc_degree_body)


def _sc_degree_body(edges, out, idx_v, ones_v, stage_v, acc):
    cid = lax.axis_index("c")
    sid = lax.axis_index("s")
    wid = cid * NS + sid

    def fill1(i, _):
        ones_v[i, :] = jnp.full((DW,), 1.0, _F32)
        return 0

    def fill0(i, _):
        stage_v[i, :] = jnp.zeros((DW,), _F32)
        return 0

    lax.fori_loop(0, CH, fill1, 0)
    lax.fori_loop(0, WCH, fill0, 0)
    for k in range(4):
        c = sid + NS * k

        @pl.when(c < NWCH)
        def _():
            pltpu.sync_copy(stage_v, acc.at[pl.ds(c * WCH, WCH)])

    plsc.subcore_barrier()

    def body(j, _):
        base = (wid + NW * j) * CH
        pltpu.sync_copy(edges.at[:, pl.ds(base, CH)], idx_v)
        pltpu.sync_copy(ones_v, acc.at[idx_v.at[1]], add=True)
        return 0

    lax.fori_loop(0, NFULL, body, 0)

    @pl.when(wid < NEXTRA)
    def _():
        base = (NW * NFULL + wid) * CH
        pltpu.sync_copy(edges.at[:, pl.ds(base, CH)], idx_v)
        pltpu.sync_copy(ones_v, acc.at[idx_v.at[1]], add=True)

    plsc.subcore_barrier()
    for k in range(4):
        c = sid + NS * k

        @pl.when(c < NWCH)
        def _():
            start = c * WCH
            pltpu.sync_copy(acc.at[pl.ds(start, WCH)], stage_v)
            pltpu.sync_copy(stage_v, out.at[cid, pl.ds(start, WCH)])


@functools.cache
def _make_sc_agg():
    return functools.partial(
        pl.kernel,
        out_type=jax.ShapeDtypeStruct((NC, N, F), _F32),
        mesh=_sc_mesh(),
        scratch_types=[
            pltpu.VMEM((2, CH), jnp.int32),
            pltpu.VMEM((2, CH), jnp.int32),
            pltpu.VMEM((CH, F), _F32),
            pltpu.VMEM((CH, F), _F32),
            pltpu.VMEM_SHARED((N, F), _F32),
            pltpu.SemaphoreType.DMA,
            pltpu.SemaphoreType.DMA,
        ],
    )(_sc_agg_body)


AWCH = 128
ANW = N // AWCH
ATAIL = N - ANW * AWCH


def _sc_agg_body(edges, hs, out, idx_a, idx_b, rows_a, rows_b, acc,
                 sem_a, sem_b):
    cid = lax.axis_index("c")
    sid = lax.axis_index("s")
    wid = cid * NS + sid
    idxs = (idx_a, idx_b)
    rows = (rows_a, rows_b)
    sems = (sem_a, sem_b)

    def zfill(i, _):
        for g in range(F // 16):
            rows_a[i, pl.ds(g * 16, 16)] = jnp.zeros((16,), _F32)
        return 0

    lax.fori_loop(0, AWCH, zfill, 0)
    for k in range(5):
        c = sid + NS * k

        @pl.when(c < ANW)
        def _():
            pltpu.sync_copy(rows_a, acc.at[pl.ds(c * AWCH, AWCH)])

    @pl.when(sid == 0)
    def _():
        pltpu.sync_copy(rows_a.at[pl.ds(0, ATAIL)],
                        acc.at[pl.ds(ANW * AWCH, ATAIL)])

    plsc.subcore_barrier()

    for b in range(2):
        pltpu.sync_copy(edges.at[:, pl.ds((wid + NW * b) * CH, CH)], idxs[b])
        pltpu.async_copy(hs.at[idxs[b].at[0]], rows[b], sems[b])

    def outer(g, _):
        for b in range(2):
            j = g * 2 + b
            pltpu.make_async_copy(hs.at[idxs[b].at[0]], rows[b], sems[b]).wait()
            pltpu.sync_copy(rows[b], acc.at[idxs[b].at[1]], add=True)

            @pl.when(j + 2 < NFULL)
            def _():
                base = (wid + NW * (j + 2)) * CH
                pltpu.sync_copy(edges.at[:, pl.ds(base, CH)], idxs[b])
                pltpu.async_copy(hs.at[idxs[b].at[0]], rows[b], sems[b])

        return 0

    lax.fori_loop(0, NFULL // 2, outer, 0)

    @pl.when(wid < NEXTRA)
    def _():
        base = (NW * NFULL + wid) * CH
        pltpu.sync_copy(edges.at[:, pl.ds(base, CH)], idx_a)
        pltpu.async_copy(hs.at[idx_a.at[0]], rows_a, sem_a).wait()
        pltpu.sync_copy(rows_a, acc.at[idx_a.at[1]], add=True)

    plsc.subcore_barrier()
    for k in range(5):
        c = sid + NS * k

        @pl.when(c < ANW)
        def _():
            start = c * AWCH
            pltpu.sync_copy(acc.at[pl.ds(start, AWCH)], rows_a)
            pltpu.sync_copy(rows_a, out.at[cid, pl.ds(start, AWCH)])

    @pl.when(sid == 0)
    def _():
        start = ANW * AWCH
        pltpu.sync_copy(acc.at[pl.ds(start, ATAIL)], rows_a.at[pl.ds(0, ATAIL)])
        pltpu.sync_copy(rows_a.at[pl.ds(0, ATAIL)],
                        out.at[cid, pl.ds(start, ATAIL)])


def _call_degree(edges):
    return _make_sc_degree()(edges)


def _call_agg(edges, hs):
    return _make_sc_agg()(edges, hs)


def _k0_body(xidx, degp, tbl, w1t, dinv_o, hs_o):
    c = xidx[:, 0:1] * 3 + xidx[:, 1:2]
    oh = (lax.broadcasted_iota(jnp.int32, (N, 16), 1) == c).astype(_F32)
    h = _mm(oh, tbl[...])
    deg = degp[0][:, 0:1] + degp[1][:, 0:1] + 1.0
    dv = 1.0 / jnp.sqrt(deg)
    dinv_o[...] = dv
    hs_o[...] = _mm(h, w1t[...]) * dv


def _k0(xidx, degp, tbl, w1t):
    return pl.pallas_call(
        _k0_body,
        out_shape=[jax.ShapeDtypeStruct((N, 1), _F32),
                   jax.ShapeDtypeStruct((N, F), _F32)],
    )(xidx, degp, tbl, w1t)


NP = 10240
P1R = 1280
NB1 = NP // P1R


def _ind_of(b2d_blk, r):
    return (b2d_blk[...] ==
            lax.broadcasted_iota(jnp.int32, (B, r), 0)).astype(_F32)


def _p1_gn_body(aggp, hs, dinv, bi, b2d, xp_o, s1_o, s2_o, cnt_o):
    i = pl.program_id(0)
    xp = dinv[...] * (aggp[0] + aggp[1] + hs[...]) + bi[...]
    xp_o[...] = xp
    ind = _ind_of(b2d, P1R)

    @pl.when(i == 0)
    def _():
        s1_o[...] = jnp.zeros_like(s1_o)
        s2_o[...] = jnp.zeros_like(s2_o)
        cnt_o[...] = jnp.zeros_like(cnt_o)

    s1_o[...] += _mm(ind, xp)
    s2_o[...] += _mm(ind, xp * xp)
    cnt_o[...] += jnp.sum(ind, axis=1, keepdims=True)


def _p1_bn_body(aggp, hs, dinv, bi, bw, bb, b2d, y_o, t1_o, t2_o, cnt_o):
    i = pl.program_id(0)
    xp = dinv[...] * (aggp[0] + aggp[1] + hs[...]) + bi[...]
    y = xp * (bw[...] / np.sqrt(1.0 + 1e-5)) + bb[...]
    y_o[...] = y
    ind = _ind_of(b2d, P1R)

    @pl.when(i == 0)
    def _():
        t1_o[...] = jnp.zeros_like(t1_o)
        t2_o[...] = jnp.zeros_like(t2_o)
        cnt_o[...] = jnp.zeros_like(cnt_o)

    t1_o[...] += _mm(ind, y)
    t2_o[...] += _mm(ind, y * y)
    cnt_o[...] += jnp.sum(ind, axis=1, keepdims=True)


def _p2_gn_body(xp, b2d, s1, s2, cnt, gw, gb, gms, y_o, t1_o, t2_o):
    i = pl.program_id(0)
    cm = jnp.maximum(cnt[...], 1.0)
    mean = s1[...] / cm
    ms = gms[...]
    var = s2[...] / cm - (2.0 * ms - ms * ms) * mean * mean
    wfac = gw[...] / jnp.sqrt(var + 1e-5)
    ind = _ind_of(b2d, P1R)
    meanb = _mm00(ind, ms * mean)
    wfacb = _mm00(ind, wfac)
    y = (xp[...] - meanb) * wfacb + gb[...]
    y_o[...] = y

    @pl.when(i == 0)
    def _():
        t1_o[...] = jnp.zeros_like(t1_o)
        t2_o[...] = jnp.zeros_like(t2_o)

    t1_o[...] += _mm(ind, y)
    t2_o[...] += _mm(ind, y * y)


def _p3_body(last, y, b2d, t1, t2, cnt, dinv, wnt, xi_o, *rest):
    cm = jnp.maximum(cnt[...], 1.0)
    mean2 = t1[...] / cm
    s = (jnp.sum(t2[...], axis=1, keepdims=True) / cm
         - jnp.sum(mean2 * mean2, axis=1, keepdims=True))
    inv = 1.0 / jnp.sqrt(1e-5 + s)
    ind = _ind_of(b2d, P1R)
    mean2b = _mm00(ind, mean2)
    invb = _mm00(ind, inv)
    xi = jnp.maximum((y[...] - mean2b) * invb, 0.0)
    xi_o[...] = xi
    if not last:
        rest[0][...] = _mm(xi, wnt[...]) * dinv[...]


_ROW = lambda lanes=F: pl.BlockSpec((P1R, lanes), lambda i: (i, 0))
_FIX = lambda shape: pl.BlockSpec(shape, lambda i: tuple(0 for _ in shape))
_B2D = pl.BlockSpec((B, P1R), lambda i: (0, i))
_ACC = lambda lanes: pl.BlockSpec((B, lanes), lambda i: (0, 0))


def _layer(mode, last, aggp, hs, dinv, b2d, p0, p1, p2, bi, wnt):
    if mode == "gn":
        xp, s1, s2, cnt = pl.pallas_call(
            _p1_gn_body,
            grid=(NB1,),
            in_specs=[pl.BlockSpec((2, P1R, F), lambda i: (0, i, 0)),
                      _ROW(), _ROW(1), _FIX((1, F)), _B2D],
            out_specs=[_ROW(), _ACC(F), _ACC(F), _ACC(1)],
            out_shape=[jax.ShapeDtypeStruct((NP, F), _F32),
                       jax.ShapeDtypeStruct((B, F), _F32),
                       jax.ShapeDtypeStruct((B, F), _F32),
                       jax.ShapeDtypeStruct((B, 1), _F32)],
        )(aggp, hs, dinv, bi, b2d)
        y, t1, t2 = pl.pallas_call(
            _p2_gn_body,
            grid=(NB1,),
            in_specs=[_ROW(), _B2D, _FIX((B, F)), _FIX((B, F)), _FIX((B, 1)),
                      _FIX((1, F)), _FIX((1, F)), _FIX((1, F))],
            out_specs=[_ROW(), _ACC(F), _ACC(F)],
            out_shape=[jax.ShapeDtypeStruct((NP, F), _F32),
                       jax.ShapeDtypeStruct((B, F), _F32),
                       jax.ShapeDtypeStruct((B, F), _F32)],
        )(xp, b2d, s1, s2, cnt, p0, p1, p2)
    else:
        y, t1, t2, cnt = pl.pallas_call(
            _p1_bn_body,
            grid=(NB1,),
            in_specs=[pl.BlockSpec((2, P1R, F), lambda i: (0, i, 0)),
                      _ROW(), _ROW(1), _FIX((1, F)), _FIX((1, F)),
                      _FIX((1, F)), _B2D],
            out_specs=[_ROW(), _ACC(F), _ACC(F), _ACC(1)],
            out_shape=[jax.ShapeDtypeStruct((NP, F), _F32),
                       jax.ShapeDtypeStruct((B, F), _F32),
                       jax.ShapeDtypeStruct((B, F), _F32),
                       jax.ShapeDtypeStruct((B, 1), _F32)],
        )(aggp, hs, dinv, bi, p0, p1, b2d)

    outs = [jax.ShapeDtypeStruct((NP, F), _F32)]
    out_specs = [_ROW()]
    if not last:
        outs.append(jax.ShapeDtypeStruct((NP, F), _F32))
        out_specs.append(_ROW())
    return pl.pallas_call(
        functools.partial(_p3_body, last),
        grid=(NB1,),
        in_specs=[_ROW(), _B2D, _FIX((B, F)), _FIX((B, F)), _FIX((B, 1)),
                  _ROW(1), _FIX((F, F))],
        out_specs=out_specs,
        out_shape=outs,
    )(y, b2d, t1, t2, cnt, dinv, wnt)


RB = 1280


def _f1_body(x1, x3, x5, ws, bs, big, l_o, lt_o, v_o):
    def w(i):
        return ws[i]

    def bv(i):
        return bs[i:i + 1, :]

    xs = _mm(x1[...], w(0)) + bv(0)
    xm = _mm(x3[...], w(1)) + bv(1)
    xl = _mm(x5[...], w(2)) + bv(2)

    def catt(xk, yq, wo, bo):
        q = _mm(yq, w(wo)) + bv(bo)
        k = _mm(xk, w(wo + 1)) + bv(bo + 1)
        v = _mm(xk, w(wo + 2)) + bv(bo + 2)
        a = jax.nn.sigmoid(q * k) * v
        o = _mm(a, w(wo + 3)) + bv(bo + 3)
        return jnp.maximum(_ln(o + yq, bv(bo + 4), bv(bo + 5)), 0.0)

    sm = catt(xs, xm, 3, 3)
    sl = catt(xs, xl, 7, 9)
    ml = catt(xm, xl, 11, 15)
    cat = jnp.concatenate([sm, sl, ml], axis=1)
    gate = jax.nn.sigmoid(_mm(cat, big[1]) + bv(29))
    fused = _mm(cat, big[0]) + bv(28)
    fused = gate * fused + (1.0 - gate) * xm
    xf = jnp.maximum(_ln(fused + xm, bv(21), bv(22)), 0.0)

    y = jnp.maximum(_mm(xf, w(15)) + bv(23), 0.0)
    kk = _mm(y, w(16)) + bv(24)
    vv = _mm(y, w(17)) + bv(25)
    qvec = _mm(bv(27), w(18)) + bv(26)
    lane = lax.broadcasted_iota(jnp.int32, (8, F), 1)
    head = lax.broadcasted_iota(jnp.int32, (8, F), 0)
    qmat = jnp.where(lane // 16 == head, qvec * 0.25, 0.0)
    l_o[...] = _mm11(kk, qmat)
    lt_o[...] = _mm11(qmat, kk)
    v_o[...] = vv


def _f1(x1, x3, x5, ws, bs, big):
    nb = NP // RB
    zmap = lambda nd: (lambda i: (0,) * nd)
    return pl.pallas_call(
        _f1_body,
        grid=(nb,),
        in_specs=[pl.BlockSpec((RB, F), lambda i: (i, 0))] * 3 + [
            pl.BlockSpec(ws.shape, zmap(3)),
            pl.BlockSpec(bs.shape, zmap(2)),
            pl.BlockSpec(big.shape, zmap(3)),
        ],
        out_specs=[pl.BlockSpec((RB, 8), lambda i: (i, 0)),
                   pl.BlockSpec((8, RB), lambda i: (0, i)),
                   pl.BlockSpec((RB, F), lambda i: (i, 0))],
        out_shape=[jax.ShapeDtypeStruct((NP, 8), _F32),
                   jax.ShapeDtypeStruct((8, NP), _F32),
                   jax.ShapeDtypeStruct((NP, F), _F32)],
    )(x1, x3, x5, ws, bs, big)


def _f2_body(L, LT, V, b2d,
             vwt, linb, vb, out1t, out1b, seed1, lin1t, lin1b,
             vw2t, vb2, out2t, out2b, lin2t, lin2b,
             lw3t, lb3, vw3t, vb3, out3t, out3b, seed2, lin3t, lin3b,
             p1t, p1b, p2t, p2b,
             mut, mub, mbw, mbb, vart, varb, vbw, vbb, epsc,
             lat_o, mu_o, var_o):
    indb = b2d[...] == lax.broadcasted_iota(jnp.int32, (B, NP), 0)
    ind = indb.astype(_F32)
    cnt = jnp.sum(ind, axis=1, keepdims=True)

    cols = []
    for h in range(8):
        row = LT[h:h + 1, :]
        cols.append(jnp.max(jnp.where(indb, row, -1e30), axis=1, keepdims=True))
    maxes = jnp.concatenate(cols, axis=1)

    mnode = _mm00(ind, maxes)
    p = jnp.exp(L[...] - mnode)
    lane = lax.broadcasted_iota(jnp.int32, (F, 8), 0)
    head = lax.broadcasted_iota(jnp.int32, (F, 8), 1)
    gmat = (lane // 16 == head).astype(_F32)
    prep = _mm11(p, gmat)
    s = _mm(ind, p)
    wv = _mm(ind, prep * V[...])
    srep = _mm11(s, gmat)
    pooled = wv / jnp.where(srep > 0.0, srep, 1.0)

    y0 = jnp.maximum(linb[...], 0.0)
    v0 = _mm(y0, vwt[...]) + vb[...]
    pooled = jnp.where(cnt > 0.0, pooled, v0)

    o1 = _mm(pooled, out1t[...]) + out1b[...] + seed1[...]
    g = o1 + jnp.maximum(_mm(o1, lin1t[...]) + lin1b[...], 0.0)

    v2 = _mm(g, vw2t[...]) + vb2[...]
    o2 = _mm(v2, out2t[...]) + out2b[...] + g
    g2 = o2 + jnp.maximum(_mm(o2, lin2t[...]) + lin2b[...], 0.0)

    y3 = jnp.maximum(_mm(g2, lw3t[...]) + lb3[...], 0.0)
    v3 = _mm(y3, vw3t[...]) + vb3[...]
    o3 = _mm(v3, out3t[...]) + out3b[...] + seed2[...]
    g3 = o3 + jnp.maximum(_mm(o3, lin3t[...]) + lin3b[...], 0.0)

    t = jnp.maximum(_mm(g3, p1t[...]) + p1b[...], 0.0)
    gg = _mm(t, p2t[...]) + p2b[...]

    bnscale = 1.0 / np.sqrt(1.0 + 1e-5)
    mu = (_mm(gg, mut[...]) + mub[...]) * bnscale * mbw[...] + mbb[...]
    var = (_mm(gg, vart[...]) + varb[...]) * bnscale * vbw[...] + vbb[...]
    lat_o[...] = epsc[...] * jnp.exp(0.5 * var) + mu
    mu_o[...] = mu
    var_o[...] = var


def _f2(L, LT, V, b2d, args):
    return pl.pallas_call(
        _f2_body,
        out_shape=[jax.ShapeDtypeStruct((B, H), _F32)] * 3,
    )(L, LT, V, b2d, *args)


def kernel(x, edge_index, batch, params):
    P = params
    x = x.astype(jnp.int32)
    edge_index = edge_index.astype(jnp.int32)
    batch = batch.astype(jnp.int32)

    def T(p):
        return p['W'].T

    def bb(p):
        return p['b'][None, :]

    tbl = jnp.zeros((16, H), _F32).at[:9].set(
        (P['node_emb'][:3, None, :] + P['chir_emb'][None, :3, :]).reshape(9, H))

    b2dp = jnp.broadcast_to(
        jnp.pad(batch, (0, NP - N), constant_values=B)[None, :], (B, NP))
    padr = lambda a: jnp.pad(a, ((0, NP - N), (0, 0)))

    degp = _call_degree(edge_index)
    dinv, hs = _k0(x, degp, tbl, T(P['gin1']))
    dinv = padr(dinv)

    xs = {}
    cur = padr(hs)
    for i in (1, 2, 3, 4, 5):
        aggp = _call_agg(edge_index, cur[:N])
        aggp = jnp.pad(aggp, ((0, 0), (0, NP - N), (0, 0)))
        last = i == 5
        if i <= 3:
            g = P['gn%d' % i]
            p0, p1, p2 = (g['weight'][None, :], g['bias'][None, :],
                          g['mean_scale'][None, :])
            mode = "gn"
        else:
            bn = P['bn%d' % i]
            p0, p1, p2 = bn['w'][None, :], bn['b'][None, :], bn['b'][None, :]
            mode = "bn"
        wnt = T(P['gin%d' % (i + 1)]) if not last else jnp.zeros((F, F), _F32)
        res = _layer(mode, last, aggp, cur, dinv, b2dp, p0, p1, p2,
                     bb(P['gin%d' % i]), wnt)
        xs[i] = res[0]
        cur = res[1] if not last else None

    fz = P['fuse']
    m1 = P['gmt']['pma1']['mab']

    def ca(c):
        return (T(fz[c]['q']), T(fz[c]['k']), T(fz[c]['v']), T(fz[c]['out']))

    ws = jnp.stack([T(fz['proj_s']), T(fz['proj_m']), T(fz['proj_l']),
                    *ca('att_sm'), *ca('att_sl'), *ca('att_ml'),
                    T(P['gmt']['pma1']['lin']),
                    m1['in_w'][F:2 * F].T, m1['in_w'][2 * F:].T,
                    m1['in_w'][:F].T])

    def cab(c):
        return (fz[c]['q']['b'], fz[c]['k']['b'], fz[c]['v']['b'],
                fz[c]['out']['b'], fz[c]['ln_w'], fz[c]['ln_b'])

    bs = jnp.stack([fz['proj_s']['b'], fz['proj_m']['b'], fz['proj_l']['b'],
                    *cab('att_sm'), *cab('att_sl'), *cab('att_ml'),
                    fz['ln_w'], fz['ln_b'],
                    P['gmt']['pma1']['lin']['b'],
                    m1['in_b'][F:2 * F], m1['in_b'][2 * F:], m1['in_b'][:F],
                    P['gmt']['pma1']['seed'][0, 0],
                    fz['final_fc']['b'], fz['gate_fc']['b']])
    big = jnp.stack([T(fz['final_fc']), T(fz['gate_fc'])])

    L, LT, V = _f1(xs[1], xs[3], xs[5], ws, bs, big)

    sab = P['gmt']['sab']
    pm2 = P['gmt']['pma2']
    m2 = pm2['mab']
    epsc = jax.random.normal(jax.random.key(123), (B, H), _F32) * 0.1
    f2_args = (
        m1['in_w'][2 * F:].T, P['gmt']['pma1']['lin']['b'][None, :],
        m1['in_b'][2 * F:][None, :],
        T(m1['out']), bb(m1['out']), P['gmt']['pma1']['seed'][0],
        T(m1['lin']), bb(m1['lin']),
        sab['in_w'][2 * F:].T, sab['in_b'][2 * F:][None, :],
        T(sab['out']), bb(sab['out']), T(sab['lin']), bb(sab['lin']),
        T(pm2['lin']), bb(pm2['lin']),
        m2['in_w'][2 * F:].T, m2['in_b'][2 * F:][None, :],
        T(m2['out']), bb(m2['out']), pm2['seed'][0],
        T(m2['lin']), bb(m2['lin']),
        T(P['proj1']), bb(P['proj1']), T(P['proj2']), bb(P['proj2']),
        T(P['mu_lin']), bb(P['mu_lin']),
        P['mu_bn']['w'][None, :], P['mu_bn']['b'][None, :],
        T(P['var_lin']), bb(P['var_lin']),
        P['var_bn']['w'][None, :], P['var_bn']['b'][None, :],
        epsc,
    )
    latent, mu, var = _f2(L, LT, V, b2dp, f2_args)
    return latent, mu, var


# speedup vs baseline: 17.2936x; 1.0363x over previous
import functools

import jax
import jax.numpy as jnp
import numpy as np
from jax import lax
from jax.experimental import pallas as pl
from jax.experimental.pallas import tpu as pltpu
from jax.experimental.pallas import tpu_sc as plsc

N = 10000
E = 320000
B = 64
F = 128
H = 64

NC = 2
NS = 16
NW = NC * NS
CH = 128
NCHUNK = E // CH
NFULL = NCHUNK // NW
NEXTRA = NCHUNK - NFULL * NW
WCH = 200
NWCH = N // WCH

_HI = jax.lax.Precision.HIGHEST
_F32 = jnp.float32


def _mm(a, b):
    return lax.dot_general(a, b, (((1,), (0,)), ((), ())),
                           precision=_HI, preferred_element_type=_F32)


def _mm11(a, b):
    return lax.dot_general(a, b, (((1,), (1,)), ((), ())),
                           precision=_HI, preferred_element_type=_F32)


def _mm00(a, b):
    return lax.dot_general(a, b, (((0,), (0,)), ((), ())),
                           precision=_HI, preferred_element_type=_F32)


def _ln(x, w, b):
    m = jnp.mean(x, axis=1, keepdims=True)
    xc = x - m
    v = jnp.mean(xc * xc, axis=1, keepdims=True)
    return xc / jnp.sqrt(v + 1e-5) * w + b


def _sc_mesh():
    return plsc.VectorSubcoreMesh(core_axis_name="c", subcore_axis_name="s",
                                  num_cores=NC, num_subcores=NS)


DW = 16


@functools.cache
def _make_sc_degree():
    return functools.partial(
        pl.kernel,
        out_type=jax.ShapeDtypeStruct((NC, N, DW), _F32),
        mesh=_sc_mesh(),
        scratch_types=[
            pltpu.VMEM((2, CH), jnp.int32),
            pltpu.VMEM((CH, DW), _F32),
            pltpu.VMEM((WCH, DW), _F32),
            pltpu.VMEM_SHARED((N, DW), _F32),
        ],
    )(_sc_degree_body)


def _sc_degree_body(edges, out, idx_v, ones_v, stage_v, acc):
    cid = lax.axis_index("c")
    sid = lax.axis_index("s")
    wid = cid * NS + sid

    def fill1(i, _):
        ones_v[i, :] = jnp.full((DW,), 1.0, _F32)
        return 0

    def fill0(i, _):
        stage_v[i, :] = jnp.zeros((DW,), _F32)
        return 0

    lax.fori_loop(0, CH, fill1, 0)
    lax.fori_loop(0, WCH, fill0, 0)
    for k in range(4):
        c = sid + NS * k

        @pl.when(c < NWCH)
        def _():
            pltpu.sync_copy(stage_v, acc.at[pl.ds(c * WCH, WCH)])

    plsc.subcore_barrier()

    def body(j, _):
        base = (wid + NW * j) * CH
        pltpu.sync_copy(edges.at[:, pl.ds(base, CH)], idx_v)
        pltpu.sync_copy(ones_v, acc.at[idx_v.at[1]], add=True)
        return 0

    lax.fori_loop(0, NFULL, body, 0)

    @pl.when(wid < NEXTRA)
    def _():
        base = (NW * NFULL + wid) * CH
        pltpu.sync_copy(edges.at[:, pl.ds(base, CH)], idx_v)
        pltpu.sync_copy(ones_v, acc.at[idx_v.at[1]], add=True)

    plsc.subcore_barrier()
    for k in range(4):
        c = sid + NS * k

        @pl.when(c < NWCH)
        def _():
            start = c * WCH
            pltpu.sync_copy(acc.at[pl.ds(start, WCH)], stage_v)
            pltpu.sync_copy(stage_v, out.at[cid, pl.ds(start, WCH)])


@functools.cache
def _make_sc_agg():
    return functools.partial(
        pl.kernel,
        out_type=jax.ShapeDtypeStruct((NC, N, F), _F32),
        mesh=_sc_mesh(),
        scratch_types=[
            pltpu.VMEM((2, CH), jnp.int32),
            pltpu.VMEM((2, CH), jnp.int32),
            pltpu.VMEM((CH, F), _F32),
            pltpu.VMEM((CH, F), _F32),
            pltpu.VMEM_SHARED((N, F), _F32),
            pltpu.SemaphoreType.DMA,
            pltpu.SemaphoreType.DMA,
        ],
    )(_sc_agg_body)


AWCH = 128
ANW = N // AWCH
ATAIL = N - ANW * AWCH


def _sc_agg_body(edges, hs, out, idx_a, idx_b, rows_a, rows_b, acc,
                 sem_a, sem_b):
    cid = lax.axis_index("c")
    sid = lax.axis_index("s")
    wid = cid * NS + sid
    idxs = (idx_a, idx_b)
    rows = (rows_a, rows_b)
    sems = (sem_a, sem_b)

    def zfill(i, _):
        for g in range(F // 16):
            rows_a[i, pl.ds(g * 16, 16)] = jnp.zeros((16,), _F32)
        return 0

    lax.fori_loop(0, AWCH, zfill, 0)
    for k in range(5):
        c = sid + NS * k

        @pl.when(c < ANW)
        def _():
            pltpu.sync_copy(rows_a, acc.at[pl.ds(c * AWCH, AWCH)])

    @pl.when(sid == 0)
    def _():
        pltpu.sync_copy(rows_a.at[pl.ds(0, ATAIL)],
                        acc.at[pl.ds(ANW * AWCH, ATAIL)])

    plsc.subcore_barrier()

    for b in range(2):
        pltpu.sync_copy(edges.at[:, pl.ds((wid + NW * b) * CH, CH)], idxs[b])
        pltpu.async_copy(hs.at[idxs[b].at[0]], rows[b], sems[b])

    def outer(g, _):
        for b in range(2):
            j = g * 2 + b
            pltpu.make_async_copy(hs.at[idxs[b].at[0]], rows[b], sems[b]).wait()
            pltpu.sync_copy(rows[b], acc.at[idxs[b].at[1]], add=True)

            @pl.when(j + 2 < NFULL)
            def _():
                base = (wid + NW * (j + 2)) * CH
                pltpu.sync_copy(edges.at[:, pl.ds(base, CH)], idxs[b])
                pltpu.async_copy(hs.at[idxs[b].at[0]], rows[b], sems[b])

        return 0

    lax.fori_loop(0, NFULL // 2, outer, 0)

    @pl.when(wid < NEXTRA)
    def _():
        base = (NW * NFULL + wid) * CH
        pltpu.sync_copy(edges.at[:, pl.ds(base, CH)], idx_a)
        pltpu.async_copy(hs.at[idx_a.at[0]], rows_a, sem_a).wait()
        pltpu.sync_copy(rows_a, acc.at[idx_a.at[1]], add=True)

    plsc.subcore_barrier()
    for k in range(5):
        c = sid + NS * k

        @pl.when(c < ANW)
        def _():
            start = c * AWCH
            pltpu.sync_copy(acc.at[pl.ds(start, AWCH)], rows_a)
            pltpu.sync_copy(rows_a, out.at[cid, pl.ds(start, AWCH)])

    @pl.when(sid == 0)
    def _():
        start = ANW * AWCH
        pltpu.sync_copy(acc.at[pl.ds(start, ATAIL)], rows_a.at[pl.ds(0, ATAIL)])
        pltpu.sync_copy(rows_a.at[pl.ds(0, ATAIL)],
                        out.at[cid, pl.ds(start, ATAIL)])


def _call_degree(edges):
    return _make_sc_degree()(edges)


def _call_agg(edges, hs):
    return _make_sc_agg()(edges, hs)


def _k0_body(xidx, degp, tbl, w1t, dinv_o, hs_o):
    c = xidx[:, 0:1] * 3 + xidx[:, 1:2]
    oh = (lax.broadcasted_iota(jnp.int32, (N, 16), 1) == c).astype(_F32)
    h = _mm(oh, tbl[...])
    deg = degp[0][:, 0:1] + degp[1][:, 0:1] + 1.0
    dv = 1.0 / jnp.sqrt(deg)
    dinv_o[...] = dv
    hs_o[...] = _mm(h, w1t[...]) * dv


def _k0(xidx, degp, tbl, w1t):
    return pl.pallas_call(
        _k0_body,
        out_shape=[jax.ShapeDtypeStruct((N, 1), _F32),
                   jax.ShapeDtypeStruct((N, F), _F32)],
    )(xidx, degp, tbl, w1t)


NP = 10240
P1R = 1280
NB1 = NP // P1R


def _ind_of(b2d_blk, r):
    return (b2d_blk[...] ==
            lax.broadcasted_iota(jnp.int32, (B, r), 0)).astype(_F32)


def _p1_gn_body(aggp, hs, dinv, bi, b2d, xp_o, s1_o, s2_o, cnt_o):
    i = pl.program_id(0)
    xp = dinv[...] * (aggp[0] + aggp[1] + hs[...]) + bi[...]
    xp_o[...] = xp
    ind = _ind_of(b2d, P1R)

    @pl.when(i == 0)
    def _():
        s1_o[...] = jnp.zeros_like(s1_o)
        s2_o[...] = jnp.zeros_like(s2_o)
        cnt_o[...] = jnp.zeros_like(cnt_o)

    s1_o[...] += _mm(ind, xp)
    s2_o[...] += _mm(ind, xp * xp)
    cnt_o[...] += jnp.sum(ind, axis=1, keepdims=True)


def _p3_body(mode, last, xp, b2d, s1, s2, cnt, g0, g1, g2, dinv, wnt,
             xi_o, *rest):
    cm = jnp.maximum(cnt[...], 1.0)
    mean = s1[...] / cm
    if mode == "gn":
        gw, gb, ms = g0[...], g1[...], g2[...]
        var = s2[...] / cm - (2.0 * ms - ms * ms) * mean * mean
        wfac = gw / jnp.sqrt(var + 1e-5)
        c = gb - ms * mean * wfac
    else:
        wfac = jnp.broadcast_to(g0[...] / np.sqrt(1.0 + 1e-5), (B, F))
        c = jnp.broadcast_to(g1[...], (B, F))
    t1 = wfac * s1[...] + cnt[...] * c
    t2 = (wfac * wfac * s2[...] + 2.0 * wfac * c * s1[...]
          + cnt[...] * c * c)
    mean2 = t1 / cm
    sp = (jnp.sum(t2, axis=1, keepdims=True) / cm
          - jnp.sum(mean2 * mean2, axis=1, keepdims=True))
    inv = 1.0 / jnp.sqrt(1e-5 + sp)
    A = inv * wfac
    C = inv * (c - mean2)
    ind = _ind_of(b2d, P1R)
    xi = jnp.maximum(xp[...] * _mm00(ind, A) + _mm00(ind, C), 0.0)
    xi_o[...] = xi
    if not last:
        rest[0][...] = _mm(xi, wnt[...]) * dinv[...]


_ROW = lambda lanes=F: pl.BlockSpec((P1R, lanes), lambda i: (i, 0))
_FIX = lambda shape: pl.BlockSpec(shape, lambda i: tuple(0 for _ in shape))
_B2D = pl.BlockSpec((B, P1R), lambda i: (0, i))
_ACC = lambda lanes: pl.BlockSpec((B, lanes), lambda i: (0, 0))


def _layer(mode, last, aggp, hs, dinv, b2d, p0, p1, p2, bi, wnt):
    xp, s1, s2, cnt = pl.pallas_call(
        _p1_gn_body,
        grid=(NB1,),
        in_specs=[pl.BlockSpec((2, P1R, F), lambda i: (0, i, 0)),
                  _ROW(), _ROW(1), _FIX((1, F)), _B2D],
        out_specs=[_ROW(), _ACC(F), _ACC(F), _ACC(1)],
        out_shape=[jax.ShapeDtypeStruct((NP, F), _F32),
                   jax.ShapeDtypeStruct((B, F), _F32),
                   jax.ShapeDtypeStruct((B, F), _F32),
                   jax.ShapeDtypeStruct((B, 1), _F32)],
    )(aggp, hs, dinv, bi, b2d)

    outs = [jax.ShapeDtypeStruct((NP, F), _F32)]
    out_specs = [_ROW()]
    if not last:
        outs.append(jax.ShapeDtypeStruct((NP, F), _F32))
        out_specs.append(_ROW())
    return pl.pallas_call(
        functools.partial(_p3_body, mode, last),
        grid=(NB1,),
        in_specs=[_ROW(), _B2D, _FIX((B, F)), _FIX((B, F)), _FIX((B, 1)),
                  _FIX((1, F)), _FIX((1, F)), _FIX((1, F)),
                  _ROW(1), _FIX((F, F))],
        out_specs=out_specs,
        out_shape=outs,
    )(xp, b2d, s1, s2, cnt, p0, p1, p2, dinv, wnt)


RB = 1280


def _f1_body(x1, x3, x5, ws, bs, big, l_o, lt_o, v_o):
    def w(i):
        return ws[i]

    def bv(i):
        return bs[i:i + 1, :]

    xs = _mm(x1[...], w(0)) + bv(0)
    xm = _mm(x3[...], w(1)) + bv(1)
    xl = _mm(x5[...], w(2)) + bv(2)

    def catt(xk, yq, wo, bo):
        q = _mm(yq, w(wo)) + bv(bo)
        k = _mm(xk, w(wo + 1)) + bv(bo + 1)
        v = _mm(xk, w(wo + 2)) + bv(bo + 2)
        a = jax.nn.sigmoid(q * k) * v
        o = _mm(a, w(wo + 3)) + bv(bo + 3)
        return jnp.maximum(_ln(o + yq, bv(bo + 4), bv(bo + 5)), 0.0)

    sm = catt(xs, xm, 3, 3)
    sl = catt(xs, xl, 7, 9)
    ml = catt(xm, xl, 11, 15)
    cat = jnp.concatenate([sm, sl, ml], axis=1)
    gate = jax.nn.sigmoid(_mm(cat, big[1]) + bv(29))
    fused = _mm(cat, big[0]) + bv(28)
    fused = gate * fused + (1.0 - gate) * xm
    xf = jnp.maximum(_ln(fused + xm, bv(21), bv(22)), 0.0)

    y = jnp.maximum(_mm(xf, w(15)) + bv(23), 0.0)
    kk = _mm(y, w(16)) + bv(24)
    vv = _mm(y, w(17)) + bv(25)
    qvec = _mm(bv(27), w(18)) + bv(26)
    lane = lax.broadcasted_iota(jnp.int32, (8, F), 1)
    head = lax.broadcasted_iota(jnp.int32, (8, F), 0)
    qmat = jnp.where(lane // 16 == head, qvec * 0.25, 0.0)
    l_o[...] = _mm11(kk, qmat)
    lt_o[...] = _mm11(qmat, kk)
    v_o[...] = vv


def _f1(x1, x3, x5, ws, bs, big):
    nb = NP // RB
    zmap = lambda nd: (lambda i: (0,) * nd)
    return pl.pallas_call(
        _f1_body,
        grid=(nb,),
        in_specs=[pl.BlockSpec((RB, F), lambda i: (i, 0))] * 3 + [
            pl.BlockSpec(ws.shape, zmap(3)),
            pl.BlockSpec(bs.shape, zmap(2)),
            pl.BlockSpec(big.shape, zmap(3)),
        ],
        out_specs=[pl.BlockSpec((RB, 8), lambda i: (i, 0)),
                   pl.BlockSpec((8, RB), lambda i: (0, i)),
                   pl.BlockSpec((RB, F), lambda i: (i, 0))],
        out_shape=[jax.ShapeDtypeStruct((NP, 8), _F32),
                   jax.ShapeDtypeStruct((8, NP), _F32),
                   jax.ShapeDtypeStruct((NP, F), _F32)],
    )(x1, x3, x5, ws, bs, big)


def _f2_body(L, LT, V, b2d,
             vwt, linb, vb, out1t, out1b, seed1, lin1t, lin1b,
             vw2t, vb2, out2t, out2b, lin2t, lin2b,
             lw3t, lb3, vw3t, vb3, out3t, out3b, seed2, lin3t, lin3b,
             p1t, p1b, p2t, p2b,
             mut, mub, mbw, mbb, vart, varb, vbw, vbb, epsc,
             lat_o, mu_o, var_o):
    indb = b2d[...] == lax.broadcasted_iota(jnp.int32, (B, NP), 0)
    ind = indb.astype(_F32)
    cnt = jnp.sum(ind, axis=1, keepdims=True)

    cols = []
    for h in range(8):
        row = LT[h:h + 1, :]
        cols.append(jnp.max(jnp.where(indb, row, -1e30), axis=1, keepdims=True))
    maxes = jnp.concatenate(cols, axis=1)

    mnode = _mm00(ind, maxes)
    p = jnp.exp(L[...] - mnode)
    lane = lax.broadcasted_iota(jnp.int32, (F, 8), 0)
    head = lax.broadcasted_iota(jnp.int32, (F, 8), 1)
    gmat = (lane // 16 == head).astype(_F32)
    prep = _mm11(p, gmat)
    s = _mm(ind, p)
    wv = _mm(ind, prep * V[...])
    srep = _mm11(s, gmat)
    pooled = wv / jnp.where(srep > 0.0, srep, 1.0)

    y0 = jnp.maximum(linb[...], 0.0)
    v0 = _mm(y0, vwt[...]) + vb[...]
    pooled = jnp.where(cnt > 0.0, pooled, v0)

    o1 = _mm(pooled, out1t[...]) + out1b[...] + seed1[...]
    g = o1 + jnp.maximum(_mm(o1, lin1t[...]) + lin1b[...], 0.0)

    v2 = _mm(g, vw2t[...]) + vb2[...]
    o2 = _mm(v2, out2t[...]) + out2b[...] + g
    g2 = o2 + jnp.maximum(_mm(o2, lin2t[...]) + lin2b[...], 0.0)

    y3 = jnp.maximum(_mm(g2, lw3t[...]) + lb3[...], 0.0)
    v3 = _mm(y3, vw3t[...]) + vb3[...]
    o3 = _mm(v3, out3t[...]) + out3b[...] + seed2[...]
    g3 = o3 + jnp.maximum(_mm(o3, lin3t[...]) + lin3b[...], 0.0)

    t = jnp.maximum(_mm(g3, p1t[...]) + p1b[...], 0.0)
    gg = _mm(t, p2t[...]) + p2b[...]

    bnscale = 1.0 / np.sqrt(1.0 + 1e-5)
    mu = (_mm(gg, mut[...]) + mub[...]) * bnscale * mbw[...] + mbb[...]
    var = (_mm(gg, vart[...]) + varb[...]) * bnscale * vbw[...] + vbb[...]
    lat_o[...] = epsc[...] * jnp.exp(0.5 * var) + mu
    mu_o[...] = mu
    var_o[...] = var


def _f2(L, LT, V, b2d, args):
    return pl.pallas_call(
        _f2_body,
        out_shape=[jax.ShapeDtypeStruct((B, H), _F32)] * 3,
    )(L, LT, V, b2d, *args)


def kernel(x, edge_index, batch, params):
    P = params
    x = x.astype(jnp.int32)
    edge_index = edge_index.astype(jnp.int32)
    batch = batch.astype(jnp.int32)

    def T(p):
        return p['W'].T

    def bb(p):
        return p['b'][None, :]

    tbl = jnp.zeros((16, H), _F32).at[:9].set(
        (P['node_emb'][:3, None, :] + P['chir_emb'][None, :3, :]).reshape(9, H))

    b2dp = jnp.broadcast_to(
        jnp.pad(batch, (0, NP - N), constant_values=B)[None, :], (B, NP))
    padr = lambda a: jnp.pad(a, ((0, NP - N), (0, 0)))

    degp = _call_degree(edge_index)
    dinv, hs = _k0(x, degp, tbl, T(P['gin1']))
    dinv = padr(dinv)

    xs = {}
    cur = padr(hs)
    for i in (1, 2, 3, 4, 5):
        aggp = _call_agg(edge_index, cur[:N])
        aggp = jnp.pad(aggp, ((0, 0), (0, NP - N), (0, 0)))
        last = i == 5
        if i <= 3:
            g = P['gn%d' % i]
            p0, p1, p2 = (g['weight'][None, :], g['bias'][None, :],
                          g['mean_scale'][None, :])
            mode = "gn"
        else:
            bn = P['bn%d' % i]
            p0, p1, p2 = bn['w'][None, :], bn['b'][None, :], bn['b'][None, :]
            mode = "bn"
        wnt = T(P['gin%d' % (i + 1)]) if not last else jnp.zeros((F, F), _F32)
        res = _layer(mode, last, aggp, cur, dinv, b2dp, p0, p1, p2,
                     bb(P['gin%d' % i]), wnt)
        xs[i] = res[0]
        cur = res[1] if not last else None

    fz = P['fuse']
    m1 = P['gmt']['pma1']['mab']

    def ca(c):
        return (T(fz[c]['q']), T(fz[c]['k']), T(fz[c]['v']), T(fz[c]['out']))

    ws = jnp.stack([T(fz['proj_s']), T(fz['proj_m']), T(fz['proj_l']),
                    *ca('att_sm'), *ca('att_sl'), *ca('att_ml'),
                    T(P['gmt']['pma1']['lin']),
                    m1['in_w'][F:2 * F].T, m1['in_w'][2 * F:].T,
                    m1['in_w'][:F].T])

    def cab(c):
        return (fz[c]['q']['b'], fz[c]['k']['b'], fz[c]['v']['b'],
                fz[c]['out']['b'], fz[c]['ln_w'], fz[c]['ln_b'])

    bs = jnp.stack([fz['proj_s']['b'], fz['proj_m']['b'], fz['proj_l']['b'],
                    *cab('att_sm'), *cab('att_sl'), *cab('att_ml'),
                    fz['ln_w'], fz['ln_b'],
                    P['gmt']['pma1']['lin']['b'],
                    m1['in_b'][F:2 * F], m1['in_b'][2 * F:], m1['in_b'][:F],
                    P['gmt']['pma1']['seed'][0, 0],
                    fz['final_fc']['b'], fz['gate_fc']['b']])
    big = jnp.stack([T(fz['final_fc']), T(fz['gate_fc'])])

    L, LT, V = _f1(xs[1], xs[3], xs[5], ws, bs, big)

    sab = P['gmt']['sab']
    pm2 = P['gmt']['pma2']
    m2 = pm2['mab']
    epsc = jax.random.normal(jax.random.key(123), (B, H), _F32) * 0.1
    f2_args = (
        m1['in_w'][2 * F:].T, P['gmt']['pma1']['lin']['b'][None, :],
        m1['in_b'][2 * F:][None, :],
        T(m1['out']), bb(m1['out']), P['gmt']['pma1']['seed'][0],
        T(m1['lin']), bb(m1['lin']),
        sab['in_w'][2 * F:].T, sab['in_b'][2 * F:][None, :],
        T(sab['out']), bb(sab['out']), T(sab['lin']), bb(sab['lin']),
        T(pm2['lin']), bb(pm2['lin']),
        m2['in_w'][2 * F:].T, m2['in_b'][2 * F:][None, :],
        T(m2['out']), bb(m2['out']), pm2['seed'][0],
        T(m2['lin']), bb(m2['lin']),
        T(P['proj1']), bb(P['proj1']), T(P['proj2']), bb(P['proj2']),
        T(P['mu_lin']), bb(P['mu_lin']),
        P['mu_bn']['w'][None, :], P['mu_bn']['b'][None, :],
        T(P['var_lin']), bb(P['var_lin']),
        P['var_bn']['w'][None, :], P['var_bn']['b'][None, :],
        epsc,
    )
    latent, mu, var = _f2(L, LT, V, b2dp, f2_args)
    return latent, mu, var
